# baseline (speedup 1.0000x reference)
import functools

import jax
import jax.numpy as jnp
from jax.experimental import pallas as pl
from jax.experimental.pallas import tpu as pltpu

N_NODES = 4096
FEATS = 256
TILE_M = 1024
NSPLIT = 2
SUB_M = TILE_M // NSPLIT


def _gcn_block(x_ref, adj0_ref, adj1_ref, w_ref, b_ref, out_ref):
    x_bf = x_ref[...].astype(jnp.bfloat16)
    w_bf = w_ref[...].astype(jnp.bfloat16)
    for j, adj_ref in enumerate((adj0_ref, adj1_ref)):
        adj_bf = adj_ref[...].astype(jnp.bfloat16)
        t = jnp.dot(adj_bf, x_bf, preferred_element_type=jnp.float32)
        out = jnp.dot(t.astype(jnp.bfloat16), w_bf,
                      preferred_element_type=jnp.float32)
        out_ref[pl.ds(j * SUB_M, SUB_M), :] = out + b_ref[...]


@functools.partial(jax.jit, static_argnames=())
def kernel(input, adj, W, b):
    n, f_in = input.shape
    f_out = W.shape[1]
    b2 = b.reshape(1, f_out)
    grid = (n // TILE_M,)

    def adj_map(j):
        return lambda i: (NSPLIT * i + j, 0)

    adj_specs = [pl.BlockSpec((SUB_M, n), adj_map(j)) for j in range(NSPLIT)]
    return pl.pallas_call(
        _gcn_block,
        grid=grid,
        in_specs=[
            pl.BlockSpec((n, f_in), lambda i: (0, 0)),
            *adj_specs,
            pl.BlockSpec((f_in, f_out), lambda i: (0, 0)),
            pl.BlockSpec((1, f_out), lambda i: (0, 0)),
        ],
        out_specs=pl.BlockSpec((TILE_M, f_out), lambda i: (i, 0)),
        out_shape=jax.ShapeDtypeStruct((n, f_out), jnp.float32),
        compiler_params=pltpu.CompilerParams(
            dimension_semantics=("parallel",),
        ),
    )(input, adj, adj, W, b2)


# speedup vs baseline: 1.2218x; 1.2218x over previous
import functools

import jax
import jax.numpy as jnp
from jax.experimental import pallas as pl
from jax.experimental.pallas import tpu as pltpu

N_NODES = 4096
FEATS = 256
TILE_M = 512


def _gcn_block(x_ref, adj_ref, w_ref, b_ref, out_ref):
    t = jnp.dot(adj_ref[...], x_ref[...],
                preferred_element_type=jnp.float32,
                precision=jax.lax.Precision.DEFAULT)
    out = jnp.dot(t, w_ref[...],
                  preferred_element_type=jnp.float32,
                  precision=jax.lax.Precision.DEFAULT)
    out_ref[...] = out + b_ref[...]


@functools.partial(jax.jit, static_argnames=())
def kernel(input, adj, W, b):
    n, f_in = input.shape
    f_out = W.shape[1]
    b2 = b.reshape(1, f_out)
    grid = (n // TILE_M,)
    return pl.pallas_call(
        _gcn_block,
        grid=grid,
        in_specs=[
            pl.BlockSpec((n, f_in), lambda i: (0, 0)),
            pl.BlockSpec((TILE_M, n), lambda i: (i, 0)),
            pl.BlockSpec((f_in, f_out), lambda i: (0, 0)),
            pl.BlockSpec((1, f_out), lambda i: (0, 0)),
        ],
        out_specs=pl.BlockSpec((TILE_M, f_out), lambda i: (i, 0)),
        out_shape=jax.ShapeDtypeStruct((n, f_out), jnp.float32),
        compiler_params=pltpu.CompilerParams(
            dimension_semantics=("parallel",),
        ),
    )(input, adj, W, b2)
